# baseline (speedup 1.0000x reference)
import functools

import jax
import jax.numpy as jnp
from jax import lax
from jax.experimental import pallas as pl
from jax.experimental.pallas import tpu as pltpu
from jax.experimental.pallas import tpu_sc as plsc

NUM_CHANNELS = 100000
EMBED_DIM = 64
BATCH = 16384
DATE_DIM = 5

NC = 2
NS = 16
L = 16
NW = NC * NS
BPW = BATCH // NW
HALF = BPW // 2
CHUNK = 128
DVEC = EMBED_DIM // L

RBT = 2048
NBLK = 25
SPLIT = NBLK * RBT

_mesh = plsc.VectorSubcoreMesh(core_axis_name="c", subcore_axis_name="s")


def _pack_body(lo_ref, hi_ref, out_ref):
    eye = jnp.eye(EMBED_DIM, dtype=jnp.float32)
    out_ref[:, 0:EMBED_DIM] = lax.dot_general(
        lo_ref[...], eye, (((0,), (0,)), ((), ())),
        precision=lax.Precision.HIGHEST, preferred_element_type=jnp.float32)
    out_ref[:, EMBED_DIM:2 * EMBED_DIM] = lax.dot_general(
        hi_ref[...], eye, (((0,), (0,)), ((), ())),
        precision=lax.Precision.HIGHEST, preferred_element_type=jnp.float32)


_pack_table = pl.pallas_call(
    _pack_body,
    out_shape=jax.ShapeDtypeStruct((SPLIT, 2 * EMBED_DIM), jnp.float32),
    grid=(NBLK,),
    in_specs=[
        pl.BlockSpec((EMBED_DIM, RBT), lambda i: (0, i)),
        pl.BlockSpec((EMBED_DIM, RBT), lambda i: (0, jnp.minimum(i + NBLK,
                                                                 2 * NBLK - 2))),
    ],
    out_specs=pl.BlockSpec((RBT, 2 * EMBED_DIM), lambda i: (i, 0)),
)


@functools.partial(
    pl.kernel,
    mesh=_mesh,
    out_type=jax.ShapeDtypeStruct((BATCH, 2 * EMBED_DIM), jnp.float32),
    scratch_types=[
        pltpu.VMEM((BPW,), jnp.int32),
        pltpu.VMEM((BPW, EMBED_DIM), jnp.float32),
        pltpu.VMEM((BPW, EMBED_DIM), jnp.float32),
        pltpu.VMEM((DATE_DIM, BPW), jnp.float32),
        pltpu.VMEM((DATE_DIM, EMBED_DIM), jnp.float32),
        pltpu.VMEM((EMBED_DIM,), jnp.float32),
        pltpu.SemaphoreType.DMA,
    ],
    compiler_params=pltpu.CompilerParams(use_tc_tiling_on_sc=False),
)
def _sc_main(date_hbm, idx_hbm, view_hbm, w_hbm, bias_hbm, out_hbm,
             idx_v, rows_v, demb_v, date_v, w_v, bias_v, gsem):
    wid = lax.axis_index("s") * NC + lax.axis_index("c")
    base = wid * BPW

    pltpu.sync_copy(idx_hbm.at[pl.ds(base, BPW)], idx_v)

    def remap_body(g, carry):
        iv = idx_v[pl.ds(g * L, L)]
        hv = jnp.where(iv >= SPLIT, 1, 0)
        idx_v[pl.ds(g * L, L)] = 2 * lax.rem(iv, SPLIT) + hv
        return carry

    lax.fori_loop(0, BPW // L, remap_body, 0)

    copies = []
    for j in range(BPW // CHUNK):
        copies.append(
            pltpu.async_copy(
                view_hbm.at[idx_v.at[pl.ds(j * CHUNK, CHUNK)]],
                rows_v.at[pl.ds(j * CHUNK, CHUNK)],
                gsem,
            )
        )

    pltpu.sync_copy(date_hbm.at[:, pl.ds(base, BPW)], date_v)
    pltpu.sync_copy(w_hbm, w_v)
    pltpu.sync_copy(bias_hbm, bias_v)

    wvec = [[w_v[k, pl.ds(d * L, L)] for d in range(DVEC)]
            for k in range(DATE_DIM)]
    bvec = [bias_v[pl.ds(d * L, L)] for d in range(DVEC)]

    def group_body(g, carry):
        sv = [date_v[k, pl.ds(g * L, L)] for k in range(DATE_DIM)]
        for r in range(L):
            b = g * L + r
            for d in range(DVEC):
                acc = bvec[d]
                for k in range(DATE_DIM):
                    acc = acc + sv[k][r] * wvec[k][d]
                demb_v[b, pl.ds(d * L, L)] = acc
        return carry

    lax.fori_loop(0, BPW // L, group_body, 0)

    pltpu.sync_copy(demb_v, out_hbm.at[pl.ds(base, BPW), pl.ds(0, EMBED_DIM)])
    for c in copies:
        c.wait()
    pltpu.sync_copy(rows_v,
                    out_hbm.at[pl.ds(base, BPW), pl.ds(EMBED_DIM, EMBED_DIM)])


def kernel(date_features, channel_ids, channel_table, date_W, date_b):
    tt = channel_table.T
    pairs = _pack_table(tt, tt)
    view = pairs.reshape(2 * SPLIT, EMBED_DIM)
    return _sc_main(date_features.T, channel_ids.astype(jnp.int32), view,
                    date_W.T, date_b)


# speedup vs baseline: 1.3373x; 1.3373x over previous
import functools

import jax
import jax.numpy as jnp
from jax import lax
from jax.experimental import pallas as pl
from jax.experimental.pallas import tpu as pltpu
from jax.experimental.pallas import tpu_sc as plsc

NUM_CHANNELS = 100000
EMBED_DIM = 64
BATCH = 16384
DATE_DIM = 5

NC = 2
NS = 16
L = 16
NW = NC * NS
BPW = BATCH // NW
HALF = BPW // 2
CHUNK = 128
DVEC = EMBED_DIM // L

RBT = 2048
NBLK = 25
SPLIT = NBLK * RBT

_mesh = plsc.VectorSubcoreMesh(core_axis_name="c", subcore_axis_name="s")


def _pack_body(lo_ref, hi_ref, out_ref):
    out_ref[:, 0:EMBED_DIM] = lo_ref[...].T
    out_ref[:, EMBED_DIM:2 * EMBED_DIM] = hi_ref[...].T


_pack_table = pl.pallas_call(
    _pack_body,
    out_shape=jax.ShapeDtypeStruct((SPLIT, 2 * EMBED_DIM), jnp.float32),
    grid=(NBLK,),
    in_specs=[
        pl.BlockSpec((EMBED_DIM, RBT), lambda i: (0, i)),
        pl.BlockSpec((EMBED_DIM, RBT), lambda i: (0, jnp.minimum(i + NBLK,
                                                                 2 * NBLK - 2))),
    ],
    out_specs=pl.BlockSpec((RBT, 2 * EMBED_DIM), lambda i: (i, 0)),
)


@functools.partial(
    pl.kernel,
    mesh=_mesh,
    out_type=jax.ShapeDtypeStruct((BATCH, 2 * EMBED_DIM), jnp.float32),
    scratch_types=[
        pltpu.VMEM((BPW,), jnp.int32),
        pltpu.VMEM((BPW, EMBED_DIM), jnp.float32),
        pltpu.VMEM((BPW, EMBED_DIM), jnp.float32),
        pltpu.VMEM((DATE_DIM, BPW), jnp.float32),
        pltpu.VMEM((DATE_DIM, EMBED_DIM), jnp.float32),
        pltpu.VMEM((EMBED_DIM,), jnp.float32),
        pltpu.SemaphoreType.DMA,
    ],
    compiler_params=pltpu.CompilerParams(use_tc_tiling_on_sc=False),
)
def _sc_main(date_hbm, idx_hbm, view_hbm, w_hbm, bias_hbm, out_hbm,
             idx_v, rows_v, demb_v, date_v, w_v, bias_v, gsem):
    wid = lax.axis_index("s") * NC + lax.axis_index("c")
    base = wid * BPW

    pltpu.sync_copy(idx_hbm.at[pl.ds(base, BPW)], idx_v)

    def remap_body(g, carry):
        iv = idx_v[pl.ds(g * L, L)]
        hv = jnp.where(iv >= SPLIT, 1, 0)
        idx_v[pl.ds(g * L, L)] = 2 * lax.rem(iv, SPLIT) + hv
        return carry

    lax.fori_loop(0, BPW // L, remap_body, 0)

    copies = []
    for j in range(BPW // CHUNK):
        copies.append(
            pltpu.async_copy(
                view_hbm.at[idx_v.at[pl.ds(j * CHUNK, CHUNK)]],
                rows_v.at[pl.ds(j * CHUNK, CHUNK)],
                gsem,
            )
        )

    pltpu.sync_copy(date_hbm.at[:, pl.ds(base, BPW)], date_v)
    pltpu.sync_copy(w_hbm, w_v)
    pltpu.sync_copy(bias_hbm, bias_v)

    wvec = [[w_v[k, pl.ds(d * L, L)] for d in range(DVEC)]
            for k in range(DATE_DIM)]
    bvec = [bias_v[pl.ds(d * L, L)] for d in range(DVEC)]

    def group_body(g, carry):
        sv = [date_v[k, pl.ds(g * L, L)] for k in range(DATE_DIM)]
        for r in range(L):
            b = g * L + r
            for d in range(DVEC):
                acc = bvec[d]
                for k in range(DATE_DIM):
                    acc = acc + sv[k][r] * wvec[k][d]
                demb_v[b, pl.ds(d * L, L)] = acc
        return carry

    lax.fori_loop(0, BPW // L, group_body, 0)

    pltpu.sync_copy(demb_v, out_hbm.at[pl.ds(base, BPW), pl.ds(0, EMBED_DIM)])
    for c in copies:
        c.wait()
    pltpu.sync_copy(rows_v,
                    out_hbm.at[pl.ds(base, BPW), pl.ds(EMBED_DIM, EMBED_DIM)])


def kernel(date_features, channel_ids, channel_table, date_W, date_b):
    tt = channel_table.T
    pairs = _pack_table(tt, tt)
    view = pairs.reshape(2 * SPLIT, EMBED_DIM)
    return _sc_main(date_features.T, channel_ids.astype(jnp.int32), view,
                    date_W.T, date_b)


# speedup vs baseline: 1.4443x; 1.0800x over previous
import functools

import jax
import jax.numpy as jnp
from jax import lax
from jax.experimental import pallas as pl
from jax.experimental.pallas import tpu as pltpu
from jax.experimental.pallas import tpu_sc as plsc

NUM_CHANNELS = 100000
EMBED_DIM = 64
BATCH = 16384
DATE_DIM = 5

NC = 2
NS = 16
L = 16
NW = NC * NS
BPW = BATCH // NW
HALF = BPW // 2
CHUNK = 128
DVEC = EMBED_DIM // L

RBT = 4096
NBLK = 13
SPLIT = NBLK * RBT

_mesh = plsc.VectorSubcoreMesh(core_axis_name="c", subcore_axis_name="s")


def _pack_body(lo_ref, hi_ref, out_ref):
    out_ref[:, 0:EMBED_DIM] = lo_ref[...].T
    out_ref[:, EMBED_DIM:2 * EMBED_DIM] = hi_ref[...].T


_pack_table = pl.pallas_call(
    _pack_body,
    out_shape=jax.ShapeDtypeStruct((SPLIT, 2 * EMBED_DIM), jnp.float32),
    grid=(NBLK,),
    in_specs=[
        pl.BlockSpec((EMBED_DIM, RBT), lambda i: (0, i)),
        pl.BlockSpec((EMBED_DIM, RBT), lambda i: (0, jnp.minimum(i + NBLK,
                                                                 2 * NBLK - 2))),
    ],
    out_specs=pl.BlockSpec((RBT, 2 * EMBED_DIM), lambda i: (i, 0)),
)


@functools.partial(
    pl.kernel,
    mesh=_mesh,
    out_type=jax.ShapeDtypeStruct((BATCH, 2 * EMBED_DIM), jnp.float32),
    scratch_types=[
        pltpu.VMEM((BPW,), jnp.int32),
        pltpu.VMEM((BPW, EMBED_DIM), jnp.float32),
        pltpu.VMEM((BPW, EMBED_DIM), jnp.float32),
        pltpu.VMEM((DATE_DIM, BPW), jnp.float32),
        pltpu.VMEM((DATE_DIM, EMBED_DIM), jnp.float32),
        pltpu.VMEM((EMBED_DIM,), jnp.float32),
        pltpu.SemaphoreType.DMA,
    ],
    compiler_params=pltpu.CompilerParams(use_tc_tiling_on_sc=False),
)
def _sc_main(date_hbm, idx_hbm, view_hbm, w_hbm, bias_hbm, out_hbm,
             idx_v, rows_v, demb_v, date_v, w_v, bias_v, gsem):
    wid = lax.axis_index("s") * NC + lax.axis_index("c")
    base = wid * BPW

    pltpu.sync_copy(idx_hbm.at[pl.ds(base, BPW)], idx_v)

    def remap_body(g, carry):
        iv = idx_v[pl.ds(g * L, L)]
        hv = jnp.where(iv >= SPLIT, 1, 0)
        idx_v[pl.ds(g * L, L)] = 2 * lax.rem(iv, SPLIT) + hv
        return carry

    lax.fori_loop(0, BPW // L, remap_body, 0)

    copies = []
    for j in range(BPW // CHUNK):
        copies.append(
            pltpu.async_copy(
                view_hbm.at[idx_v.at[pl.ds(j * CHUNK, CHUNK)]],
                rows_v.at[pl.ds(j * CHUNK, CHUNK)],
                gsem,
            )
        )

    pltpu.sync_copy(date_hbm.at[:, pl.ds(base, BPW)], date_v)
    pltpu.sync_copy(w_hbm, w_v)
    pltpu.sync_copy(bias_hbm, bias_v)

    wvec = [[w_v[k, pl.ds(d * L, L)] for d in range(DVEC)]
            for k in range(DATE_DIM)]
    bvec = [bias_v[pl.ds(d * L, L)] for d in range(DVEC)]

    def group_body(g, carry):
        sv = [date_v[k, pl.ds(g * L, L)] for k in range(DATE_DIM)]
        for r in range(L):
            b = g * L + r
            for d in range(DVEC):
                acc = bvec[d]
                for k in range(DATE_DIM):
                    acc = acc + sv[k][r] * wvec[k][d]
                demb_v[b, pl.ds(d * L, L)] = acc
        return carry

    lax.fori_loop(0, BPW // L, group_body, 0)

    pltpu.sync_copy(demb_v, out_hbm.at[pl.ds(base, BPW), pl.ds(0, EMBED_DIM)])
    for c in copies:
        c.wait()
    pltpu.sync_copy(rows_v,
                    out_hbm.at[pl.ds(base, BPW), pl.ds(EMBED_DIM, EMBED_DIM)])


def kernel(date_features, channel_ids, channel_table, date_W, date_b):
    tt = channel_table.T
    pairs = _pack_table(tt, tt)
    view = pairs.reshape(2 * SPLIT, EMBED_DIM)
    return _sc_main(date_features.T, channel_ids.astype(jnp.int32), view,
                    date_W.T, date_b)


# speedup vs baseline: 1.4743x; 1.0207x over previous
import functools

import jax
import jax.numpy as jnp
from jax import lax
from jax.experimental import pallas as pl
from jax.experimental.pallas import tpu as pltpu
from jax.experimental.pallas import tpu_sc as plsc

NUM_CHANNELS = 100000
EMBED_DIM = 64
BATCH = 16384
DATE_DIM = 5

NC = 2
NS = 16
L = 16
NW = NC * NS
BPW = BATCH // NW
HALF = BPW // 2
CHUNK = 128
DVEC = EMBED_DIM // L

RBT = 8192
NBLK = 7
SPLIT = NBLK * RBT

_mesh = plsc.VectorSubcoreMesh(core_axis_name="c", subcore_axis_name="s")


def _pack_body(lo_ref, hi_ref, out_ref):
    out_ref[:, 0:EMBED_DIM] = lo_ref[...].T
    out_ref[:, EMBED_DIM:2 * EMBED_DIM] = hi_ref[...].T


_pack_table = pl.pallas_call(
    _pack_body,
    out_shape=jax.ShapeDtypeStruct((SPLIT, 2 * EMBED_DIM), jnp.float32),
    grid=(NBLK,),
    in_specs=[
        pl.BlockSpec((EMBED_DIM, RBT), lambda i: (0, i)),
        pl.BlockSpec((EMBED_DIM, RBT), lambda i: (0, jnp.minimum(i + NBLK,
                                                                 2 * NBLK - 2))),
    ],
    out_specs=pl.BlockSpec((RBT, 2 * EMBED_DIM), lambda i: (i, 0)),
)


@functools.partial(
    pl.kernel,
    mesh=_mesh,
    out_type=jax.ShapeDtypeStruct((BATCH, 2 * EMBED_DIM), jnp.float32),
    scratch_types=[
        pltpu.VMEM((BPW,), jnp.int32),
        pltpu.VMEM((BPW, EMBED_DIM), jnp.float32),
        pltpu.VMEM((BPW, EMBED_DIM), jnp.float32),
        pltpu.VMEM((DATE_DIM, BPW), jnp.float32),
        pltpu.VMEM((DATE_DIM, EMBED_DIM), jnp.float32),
        pltpu.VMEM((EMBED_DIM,), jnp.float32),
        pltpu.SemaphoreType.DMA,
    ],
    compiler_params=pltpu.CompilerParams(use_tc_tiling_on_sc=False),
)
def _sc_main(date_hbm, idx_hbm, view_hbm, w_hbm, bias_hbm, out_hbm,
             idx_v, rows_v, demb_v, date_v, w_v, bias_v, gsem):
    wid = lax.axis_index("s") * NC + lax.axis_index("c")
    base = wid * BPW

    pltpu.sync_copy(idx_hbm.at[pl.ds(base, BPW)], idx_v)

    def remap_body(g, carry):
        iv = idx_v[pl.ds(g * L, L)]
        hv = jnp.where(iv >= SPLIT, 1, 0)
        idx_v[pl.ds(g * L, L)] = 2 * lax.rem(iv, SPLIT) + hv
        return carry

    lax.fori_loop(0, BPW // L, remap_body, 0)

    copies = []
    for j in range(BPW // CHUNK):
        copies.append(
            pltpu.async_copy(
                view_hbm.at[idx_v.at[pl.ds(j * CHUNK, CHUNK)]],
                rows_v.at[pl.ds(j * CHUNK, CHUNK)],
                gsem,
            )
        )

    pltpu.sync_copy(date_hbm.at[:, pl.ds(base, BPW)], date_v)
    pltpu.sync_copy(w_hbm, w_v)
    pltpu.sync_copy(bias_hbm, bias_v)

    wvec = [[w_v[k, pl.ds(d * L, L)] for d in range(DVEC)]
            for k in range(DATE_DIM)]
    bvec = [bias_v[pl.ds(d * L, L)] for d in range(DVEC)]

    def group_body(g, carry):
        sv = [date_v[k, pl.ds(g * L, L)] for k in range(DATE_DIM)]
        for r in range(L):
            b = g * L + r
            for d in range(DVEC):
                acc = bvec[d]
                for k in range(DATE_DIM):
                    acc = acc + sv[k][r] * wvec[k][d]
                demb_v[b, pl.ds(d * L, L)] = acc
        return carry

    lax.fori_loop(0, BPW // L, group_body, 0)

    pltpu.sync_copy(demb_v, out_hbm.at[pl.ds(base, BPW), pl.ds(0, EMBED_DIM)])
    for c in copies:
        c.wait()
    pltpu.sync_copy(rows_v,
                    out_hbm.at[pl.ds(base, BPW), pl.ds(EMBED_DIM, EMBED_DIM)])


def kernel(date_features, channel_ids, channel_table, date_W, date_b):
    tt = channel_table.T
    pairs = _pack_table(tt, tt)
    view = pairs.reshape(2 * SPLIT, EMBED_DIM)
    return _sc_main(date_features.T, channel_ids.astype(jnp.int32), view,
                    date_W.T, date_b)
